# baseline (speedup 1.0000x reference)
import functools

import jax
import jax.numpy as jnp
from jax import lax
from jax.experimental import pallas as pl
from jax.experimental.pallas import tpu as pltpu
from jax.experimental.pallas import tpu_sc as plsc

N_NODES = 10000
N_EDGES = 160000
D_EDGE = 16
LANES = 16
N_WORKERS = 16
EPW = N_EDGES // N_WORKERS
NCHUNK = 5
CE = EPW // NCHUNK
CG = CE // LANES
NPAD = 10240
SEG_PW = NPAD // N_WORKERS
NEG_INF = float("-inf")


def _sc_body(ef_hbm, src_hbm, w_hbm, out_hbm,
             w_v, ef_v, a_v, src_v, ex_v, pmax_v, glob_v, red_v, gseg_v,
             kbuf, vbuf, accb,
             pmax_sh, gmax_sh, den_sh):
    wid = lax.axis_index("s")
    base_e = wid * EPW
    pltpu.sync_copy(w_hbm, w_v)
    pltpu.sync_copy(src_hbm.at[pl.ds(base_e, EPW)], src_v)

    wvec = w_v[...]
    neg = jnp.full((LANES,), NEG_INF, jnp.float32)
    iot = lax.iota(jnp.int32, LANES)
    iot16 = iot * 16

    def init_body(i, _):
        pmax_v[pl.ds(i * LANES, LANES)] = neg
        return _
    lax.fori_loop(0, NPAD // LANES, init_body, None)

    for c in range(NCHUNK):
        pltpu.sync_copy(
            ef_hbm.at[pl.ds((base_e + c * CE) * D_EDGE, CE * D_EDGE)], ef_v)

        def phase_a(g, _):
            b = c * CE + g * LANES
            s16 = src_v[pl.ds(b, LANES)]
            base16 = iot16 + g * 256
            acc = jnp.zeros((LANES,), jnp.float32)
            for f in range(D_EDGE):
                col = plsc.load_gather(ef_v, [base16 + f])
                acc = acc + col * wvec[f]
            a_v[pl.ds(b, LANES)] = acc
            sk, sv = plsc.sort_key_val(s16, acc)
            kbuf[...] = sk
            m = sv
            for k in (1, 2, 4, 8):
                j = jnp.maximum(iot - k, 0)
                vbuf[...] = m
                pm = plsc.load_gather(vbuf, [j])
                ps = plsc.load_gather(kbuf, [j])
                take = (ps == sk) & (iot >= k)
                m = jnp.where(take, jnp.maximum(m, pm), m)
            ns = plsc.load_gather(kbuf, [jnp.minimum(iot + 1, LANES - 1)])
            last = (ns != sk) | (iot == LANES - 1)
            old = plsc.load_gather(pmax_v, [sk])
            plsc.store_scatter(pmax_v, [sk], jnp.maximum(old, m), mask=last)
            return _
        lax.fori_loop(0, CG, phase_a, None)

    pltpu.sync_copy(pmax_v, pmax_sh.at[wid])
    plsc.subcore_barrier()
    seg_lo = wid * SEG_PW
    pltpu.sync_copy(pmax_sh.at[:, pl.ds(seg_lo, SEG_PW)], red_v)

    def red_body(j, _):
        cc = j * LANES
        m = red_v[0, pl.ds(cc, LANES)]
        for r in range(1, N_WORKERS):
            m = jnp.maximum(m, red_v[r, pl.ds(cc, LANES)])
        gseg_v[pl.ds(cc, LANES)] = m
        return _
    lax.fori_loop(0, SEG_PW // LANES, red_body, None)
    pltpu.sync_copy(gseg_v, gmax_sh.at[pl.ds(seg_lo, SEG_PW)])

    zeros = jnp.zeros((LANES,), jnp.float32)

    def zero_body(j, _):
        gseg_v[pl.ds(j * LANES, LANES)] = zeros
        return _
    lax.fori_loop(0, SEG_PW // LANES, zero_body, None)
    pltpu.sync_copy(gseg_v, den_sh.at[pl.ds(seg_lo, SEG_PW)])
    plsc.subcore_barrier()
    pltpu.sync_copy(gmax_sh, glob_v)

    def phase_b(i, _):
        b = i * LANES
        s16 = src_v[pl.ds(b, LANES)]
        a16 = a_v[pl.ds(b, LANES)]
        mx = plsc.load_gather(glob_v, [s16])
        ex_v[pl.ds(b, LANES)] = jnp.exp(a16 - mx)
        return _
    lax.fori_loop(0, EPW // LANES, phase_b, None)
    pltpu.sync_copy(ex_v, den_sh.at[src_v], add=True)
    plsc.subcore_barrier()
    pltpu.sync_copy(den_sh, glob_v)

    def phase_c(i, acc):
        b = i * LANES
        s16 = src_v[pl.ds(b, LANES)]
        e16 = ex_v[pl.ds(b, LANES)]
        d16 = plsc.load_gather(glob_v, [s16])
        return acc + e16 / d16
    acc = lax.fori_loop(0, EPW // LANES, phase_c,
                        jnp.zeros((LANES,), jnp.float32))
    accb[...] = acc
    pltpu.sync_copy(accb, out_hbm.at[wid])


_sc_softmax_partials = functools.partial(
    pl.kernel,
    mesh=plsc.VectorSubcoreMesh(core_axis_name="c", subcore_axis_name="s",
                                num_cores=1),
    compiler_params=pltpu.CompilerParams(needs_layout_passes=False),
    out_type=jax.ShapeDtypeStruct((N_WORKERS, LANES), jnp.float32),
    scratch_types=[
        pltpu.VMEM((LANES,), jnp.float32),
        pltpu.VMEM((CE * D_EDGE,), jnp.float32),
        pltpu.VMEM((EPW,), jnp.float32),
        pltpu.VMEM((EPW,), jnp.int32),
        pltpu.VMEM((EPW,), jnp.float32),
        pltpu.VMEM((NPAD,), jnp.float32),
        pltpu.VMEM((NPAD,), jnp.float32),
        pltpu.VMEM((N_WORKERS, SEG_PW), jnp.float32),
        pltpu.VMEM((SEG_PW,), jnp.float32),
        pltpu.VMEM((LANES,), jnp.int32),
        pltpu.VMEM((LANES,), jnp.float32),
        pltpu.VMEM((LANES,), jnp.float32),
        pltpu.VMEM_SHARED((N_WORKERS, NPAD), jnp.float32),
        pltpu.VMEM_SHARED((NPAD,), jnp.float32),
        pltpu.VMEM_SHARED((NPAD,), jnp.float32),
    ],
)(_sc_body)


def _h_body(p_ref, x_ref, o_ref):
    o_ref[...] = x_ref[...] + 0.0 * jnp.sum(p_ref[...])


def kernel(node_features, edge_features, edge_index, W_attn):
    src = edge_index[0].astype(jnp.int32)
    ef1d = edge_features.reshape(-1)
    w16 = W_attn.reshape(D_EDGE)
    partials = _sc_softmax_partials(ef1d, src, w16)

    rows, cols = node_features.shape
    blk = 2000
    h = pl.pallas_call(
        _h_body,
        grid=(rows // blk,),
        in_specs=[
            pl.BlockSpec((N_WORKERS, LANES), lambda i: (0, 0)),
            pl.BlockSpec((blk, cols), lambda i: (i, 0)),
        ],
        out_specs=pl.BlockSpec((blk, cols), lambda i: (i, 0)),
        out_shape=jax.ShapeDtypeStruct(node_features.shape,
                                       node_features.dtype),
    )(partials, node_features)
    return h


# speedup vs baseline: 1.1061x; 1.1061x over previous
import functools

import jax
import jax.numpy as jnp
from jax import lax
from jax.experimental import pallas as pl
from jax.experimental.pallas import tpu as pltpu
from jax.experimental.pallas import tpu_sc as plsc

N_NODES = 10000
N_EDGES = 160000
D_EDGE = 16
LANES = 16
N_WORKERS = 16
EPW = N_EDGES // N_WORKERS
NPAD = 10240
SEG_PW = NPAD // N_WORKERS
NEG_INF = float("-inf")


def _dot_body(w_ref, x_ref, o_ref):
    o_ref[...] = lax.dot_general(
        w_ref[...], x_ref[...], (((0,), (1,)), ((), ())),
        preferred_element_type=jnp.float32)


def _edge_logits_t(edge_features, W_attn):
    w8 = jnp.tile(W_attn, (1, 8))
    blk = 16000
    return pl.pallas_call(
        _dot_body,
        grid=(N_EDGES // blk,),
        in_specs=[
            pl.BlockSpec((D_EDGE, 8), lambda i: (0, 0)),
            pl.BlockSpec((blk, D_EDGE), lambda i: (i, 0)),
        ],
        out_specs=pl.BlockSpec((8, blk), lambda i: (0, i)),
        out_shape=jax.ShapeDtypeStruct((8, N_EDGES), jnp.float32),
    )(w8, edge_features)


def _sc_body(a_hbm, src_hbm, out_hbm,
             a_v, src_v, ex_v, pmax_v, glob_v, red_v, gseg_v,
             kbuf, vbuf, accb,
             pmax_sh, gmax_sh, den_sh):
    wid = lax.axis_index("s")
    base_e = wid * EPW
    pltpu.sync_copy(a_hbm.at[pl.ds(base_e, EPW)], a_v)
    pltpu.sync_copy(src_hbm.at[pl.ds(base_e, EPW)], src_v)

    neg = jnp.full((LANES,), NEG_INF, jnp.float32)
    iot = lax.iota(jnp.int32, LANES)

    def init_body(i, _):
        pmax_v[pl.ds(i * LANES, LANES)] = neg
        return _
    lax.fori_loop(0, NPAD // LANES, init_body, None)

    def phase_a(i, _):
        b = i * LANES
        s16 = src_v[pl.ds(b, LANES)]
        a16 = a_v[pl.ds(b, LANES)]
        sk, sv = plsc.sort_key_val(s16, a16)
        kbuf[...] = sk
        m = sv
        for k in (1, 2, 4, 8):
            j = jnp.maximum(iot - k, 0)
            vbuf[...] = m
            pm = plsc.load_gather(vbuf, [j])
            ps = plsc.load_gather(kbuf, [j])
            take = (ps == sk) & (iot >= k)
            m = jnp.where(take, jnp.maximum(m, pm), m)
        ns = plsc.load_gather(kbuf, [jnp.minimum(iot + 1, LANES - 1)])
        last = (ns != sk) | (iot == LANES - 1)
        old = plsc.load_gather(pmax_v, [sk])
        plsc.store_scatter(pmax_v, [sk], jnp.maximum(old, m), mask=last)
        return _
    lax.fori_loop(0, EPW // LANES, phase_a, None)

    pltpu.sync_copy(pmax_v, pmax_sh.at[wid])
    plsc.subcore_barrier()
    seg_lo = wid * SEG_PW
    pltpu.sync_copy(pmax_sh.at[:, pl.ds(seg_lo, SEG_PW)], red_v)

    def red_body(j, _):
        cc = j * LANES
        m = red_v[0, pl.ds(cc, LANES)]
        for r in range(1, N_WORKERS):
            m = jnp.maximum(m, red_v[r, pl.ds(cc, LANES)])
        gseg_v[pl.ds(cc, LANES)] = m
        return _
    lax.fori_loop(0, SEG_PW // LANES, red_body, None)
    pltpu.sync_copy(gseg_v, gmax_sh.at[pl.ds(seg_lo, SEG_PW)])

    zeros = jnp.zeros((LANES,), jnp.float32)

    def zero_body(j, _):
        gseg_v[pl.ds(j * LANES, LANES)] = zeros
        return _
    lax.fori_loop(0, SEG_PW // LANES, zero_body, None)
    pltpu.sync_copy(gseg_v, den_sh.at[pl.ds(seg_lo, SEG_PW)])
    plsc.subcore_barrier()
    pltpu.sync_copy(gmax_sh, glob_v)

    def phase_b(i, _):
        b = i * LANES
        s16 = src_v[pl.ds(b, LANES)]
        a16 = a_v[pl.ds(b, LANES)]
        mx = plsc.load_gather(glob_v, [s16])
        ex_v[pl.ds(b, LANES)] = jnp.exp(a16 - mx)
        return _
    lax.fori_loop(0, EPW // LANES, phase_b, None)
    pltpu.sync_copy(ex_v, den_sh.at[src_v], add=True)
    plsc.subcore_barrier()
    pltpu.sync_copy(den_sh, glob_v)

    def phase_c(i, acc):
        b = i * LANES
        s16 = src_v[pl.ds(b, LANES)]
        e16 = ex_v[pl.ds(b, LANES)]
        d16 = plsc.load_gather(glob_v, [s16])
        return acc + e16 / d16
    acc = lax.fori_loop(0, EPW // LANES, phase_c,
                        jnp.zeros((LANES,), jnp.float32))
    accb[...] = acc
    pltpu.sync_copy(accb, out_hbm.at[wid])


_sc_softmax_partials = functools.partial(
    pl.kernel,
    mesh=plsc.VectorSubcoreMesh(core_axis_name="c", subcore_axis_name="s",
                                num_cores=1),
    compiler_params=pltpu.CompilerParams(needs_layout_passes=False),
    out_type=jax.ShapeDtypeStruct((N_WORKERS, LANES), jnp.float32),
    scratch_types=[
        pltpu.VMEM((EPW,), jnp.float32),
        pltpu.VMEM((EPW,), jnp.int32),
        pltpu.VMEM((EPW,), jnp.float32),
        pltpu.VMEM((NPAD,), jnp.float32),
        pltpu.VMEM((NPAD,), jnp.float32),
        pltpu.VMEM((N_WORKERS, SEG_PW), jnp.float32),
        pltpu.VMEM((SEG_PW,), jnp.float32),
        pltpu.VMEM((LANES,), jnp.int32),
        pltpu.VMEM((LANES,), jnp.float32),
        pltpu.VMEM((LANES,), jnp.float32),
        pltpu.VMEM_SHARED((N_WORKERS, NPAD), jnp.float32),
        pltpu.VMEM_SHARED((NPAD,), jnp.float32),
        pltpu.VMEM_SHARED((NPAD,), jnp.float32),
    ],
)(_sc_body)


def _h_body(p_ref, x_ref, o_ref):
    o_ref[...] = x_ref[...] + 0.0 * jnp.sum(p_ref[...])


def kernel(node_features, edge_features, edge_index, W_attn):
    src = edge_index[0].astype(jnp.int32)
    a_t = _edge_logits_t(edge_features, W_attn)
    a = a_t[0]
    partials = _sc_softmax_partials(a, src)

    rows, cols = node_features.shape
    blk = 2000
    h = pl.pallas_call(
        _h_body,
        grid=(rows // blk,),
        in_specs=[
            pl.BlockSpec((N_WORKERS, LANES), lambda i: (0, 0)),
            pl.BlockSpec((blk, cols), lambda i: (i, 0)),
        ],
        out_specs=pl.BlockSpec((blk, cols), lambda i: (i, 0)),
        out_shape=jax.ShapeDtypeStruct(node_features.shape,
                                       node_features.dtype),
    )(partials, node_features)
    return h


# speedup vs baseline: 2.2212x; 2.0081x over previous
import functools

import jax
import jax.numpy as jnp
from jax import lax
from jax.experimental import pallas as pl
from jax.experimental.pallas import tpu as pltpu
from jax.experimental.pallas import tpu_sc as plsc

N_NODES = 10000
N_EDGES = 160000
D_EDGE = 16
LANES = 16
N_WORKERS = 16
EPW = N_EDGES // N_WORKERS
NPAD = 10240
SEG_PW = NPAD // N_WORKERS
NEG_INF = float("-inf")


def _prep_body(w_ref, x_ref, idx_ref, a_ref, src_ref):
    y = lax.dot_general(w_ref[...], x_ref[...], (((0,), (0,)), ((), ())),
                        preferred_element_type=jnp.float32)
    a_ref[...] = y[0]
    src_ref[...] = idx_ref[0]


def _edge_prep(edge_features, edge_index, W_attn):
    eft = edge_features.T
    return pl.pallas_call(
        _prep_body,
        out_shape=[
            jax.ShapeDtypeStruct((N_EDGES,), jnp.float32),
            jax.ShapeDtypeStruct((N_EDGES,), jnp.int32),
        ],
    )(W_attn, eft, edge_index)


def _sc_body(a_hbm, src_hbm, out_hbm,
             a_v, src_v, ex_v, pmax_v, glob_v, red_v, gseg_v,
             kbuf, vbuf, accb,
             pmax_sh, gmax_sh, den_sh):
    wid = lax.axis_index("s")
    base_e = wid * EPW
    pltpu.sync_copy(a_hbm.at[pl.ds(base_e, EPW)], a_v)
    pltpu.sync_copy(src_hbm.at[pl.ds(base_e, EPW)], src_v)

    neg = jnp.full((LANES,), NEG_INF, jnp.float32)
    iot = lax.iota(jnp.int32, LANES)

    def init_body(i, _):
        pmax_v[pl.ds(i * LANES, LANES)] = neg
        return _
    lax.fori_loop(0, NPAD // LANES, init_body, None)

    def phase_a(i, _):
        b = i * LANES
        s16 = src_v[pl.ds(b, LANES)]
        a16 = a_v[pl.ds(b, LANES)]
        sk, sv = plsc.sort_key_val(s16, a16)
        kbuf[...] = sk
        m = sv
        for k in (1, 2, 4, 8):
            j = jnp.maximum(iot - k, 0)
            vbuf[...] = m
            pm = plsc.load_gather(vbuf, [j])
            ps = plsc.load_gather(kbuf, [j])
            take = (ps == sk) & (iot >= k)
            m = jnp.where(take, jnp.maximum(m, pm), m)
        ns = plsc.load_gather(kbuf, [jnp.minimum(iot + 1, LANES - 1)])
        last = (ns != sk) | (iot == LANES - 1)
        old = plsc.load_gather(pmax_v, [sk])
        plsc.store_scatter(pmax_v, [sk], jnp.maximum(old, m), mask=last)
        return _
    lax.fori_loop(0, EPW // LANES, phase_a, None)

    pltpu.sync_copy(pmax_v, pmax_sh.at[wid])
    plsc.subcore_barrier()
    seg_lo = wid * SEG_PW
    pltpu.sync_copy(pmax_sh.at[:, pl.ds(seg_lo, SEG_PW)], red_v)

    def red_body(j, _):
        cc = j * LANES
        m = red_v[0, pl.ds(cc, LANES)]
        for r in range(1, N_WORKERS):
            m = jnp.maximum(m, red_v[r, pl.ds(cc, LANES)])
        gseg_v[pl.ds(cc, LANES)] = m
        return _
    lax.fori_loop(0, SEG_PW // LANES, red_body, None)
    pltpu.sync_copy(gseg_v, gmax_sh.at[pl.ds(seg_lo, SEG_PW)])

    zeros = jnp.zeros((LANES,), jnp.float32)

    def zero_body(j, _):
        gseg_v[pl.ds(j * LANES, LANES)] = zeros
        return _
    lax.fori_loop(0, SEG_PW // LANES, zero_body, None)
    pltpu.sync_copy(gseg_v, den_sh.at[pl.ds(seg_lo, SEG_PW)])
    plsc.subcore_barrier()
    pltpu.sync_copy(gmax_sh, glob_v)

    def phase_b(i, _):
        b = i * LANES
        s16 = src_v[pl.ds(b, LANES)]
        a16 = a_v[pl.ds(b, LANES)]
        mx = plsc.load_gather(glob_v, [s16])
        ex_v[pl.ds(b, LANES)] = jnp.exp(a16 - mx)
        return _
    lax.fori_loop(0, EPW // LANES, phase_b, None)
    pltpu.sync_copy(ex_v, den_sh.at[src_v], add=True)
    plsc.subcore_barrier()
    pltpu.sync_copy(den_sh, glob_v)

    def phase_c(i, acc):
        b = i * LANES
        s16 = src_v[pl.ds(b, LANES)]
        e16 = ex_v[pl.ds(b, LANES)]
        d16 = plsc.load_gather(glob_v, [s16])
        return acc + e16 / d16
    acc = lax.fori_loop(0, EPW // LANES, phase_c,
                        jnp.zeros((LANES,), jnp.float32))
    accb[...] = acc
    pltpu.sync_copy(accb, out_hbm.at[wid])


_sc_softmax_partials = functools.partial(
    pl.kernel,
    mesh=plsc.VectorSubcoreMesh(core_axis_name="c", subcore_axis_name="s",
                                num_cores=1),
    compiler_params=pltpu.CompilerParams(needs_layout_passes=False),
    out_type=jax.ShapeDtypeStruct((N_WORKERS, LANES), jnp.float32),
    scratch_types=[
        pltpu.VMEM((EPW,), jnp.float32),
        pltpu.VMEM((EPW,), jnp.int32),
        pltpu.VMEM((EPW,), jnp.float32),
        pltpu.VMEM((NPAD,), jnp.float32),
        pltpu.VMEM((NPAD,), jnp.float32),
        pltpu.VMEM((N_WORKERS, SEG_PW), jnp.float32),
        pltpu.VMEM((SEG_PW,), jnp.float32),
        pltpu.VMEM((LANES,), jnp.int32),
        pltpu.VMEM((LANES,), jnp.float32),
        pltpu.VMEM((LANES,), jnp.float32),
        pltpu.VMEM_SHARED((N_WORKERS, NPAD), jnp.float32),
        pltpu.VMEM_SHARED((NPAD,), jnp.float32),
        pltpu.VMEM_SHARED((NPAD,), jnp.float32),
    ],
)(_sc_body)


def _h_body(p_ref, x_ref, o_ref):
    o_ref[...] = x_ref[...] + 0.0 * jnp.sum(p_ref[...])


def kernel(node_features, edge_features, edge_index, W_attn):
    a, src = _edge_prep(edge_features, edge_index.astype(jnp.int32), W_attn)
    partials = _sc_softmax_partials(a, src)

    rows, cols = node_features.shape
    blk = 2000
    h = pl.pallas_call(
        _h_body,
        grid=(rows // blk,),
        in_specs=[
            pl.BlockSpec((N_WORKERS, LANES), lambda i: (0, 0)),
            pl.BlockSpec((blk, cols), lambda i: (i, 0)),
        ],
        out_specs=pl.BlockSpec((blk, cols), lambda i: (i, 0)),
        out_shape=jax.ShapeDtypeStruct(node_features.shape,
                                       node_features.dtype),
    )(partials, node_features)
    return h
